# baseline (speedup 1.0000x reference)
import jax
import jax.numpy as jnp
from jax.experimental import pallas as pl
from jax.experimental.pallas import tpu as pltpu

_TK = 256


def _lstm_kernel(words_ref,
                 emb_hbm,
                 wih_hbm,
                 whh_hbm,
                 wout_hbm,
                 bg_ref,
                 bout_ref,
                 h0_ref,
                 c0_ref,
                 scores_hbm,
                 h_out_ref,
                 c_out_ref,
                 xbuf,
                 wih_v,
                 whh_v,
                 wout_v,
                 whh16,
                 xproj,
                 hbuf,
                 sbuf,
                 sem_rows, sem_wih, sem_whh, sem_wout, sem_out):
    T = scores_hbm.shape[0]
    H = h0_ref.shape[-1]
    V = emb_hbm.shape[0]
    KT = H // _TK

    row_cps = [
        pltpu.make_async_copy(
            emb_hbm.at[pl.ds(jnp.minimum(jnp.maximum(words_ref[t], 0), V - 1),
                             1), :],
            xbuf.at[pl.ds(t, 1), :], sem_rows)
        for t in range(T)
    ]
    for cp in row_cps:
        cp.start()
    wih_cp = pltpu.make_async_copy(wih_hbm, wih_v, sem_wih)
    wih_cp.start()
    whh_cp = pltpu.make_async_copy(whh_hbm, whh_v, sem_whh)
    whh_cp.start()
    wout_cp = pltpu.make_async_copy(wout_hbm, wout_v, sem_wout)

    for cp in row_cps:
        cp.wait()

    xb16 = xbuf[...].astype(jnp.bfloat16)
    msr_state = [0, 0]

    def mm_tiles(lhs16_full, w_ref, n_tile, mxu, base_addr, m):
        for k in range(KT):
            tile = w_ref[k * _TK:(k + 1) * _TK,
                         n_tile * _TK:(n_tile + 1) * _TK].astype(jnp.bfloat16)
            msr = msr_state[mxu]
            msr_state[mxu] = msr ^ 1
            pltpu.matmul_push_rhs(tile, msr, mxu)
            pltpu.matmul_acc_lhs(base_addr,
                                 lhs16_full[:, k * _TK:(k + 1) * _TK],
                                 mxu, load_staged_rhs=msr)

    NX = 4 * H // _TK
    wih_cp.wait()
    for n in range(NX):
        mm_tiles(xb16, wih_v, n, n % 2, 16 * (n // 2), T)
    for n in range(NX):
        part = pltpu.matmul_pop(16 * (n // 2), (T, _TK), jnp.float32, n % 2)
        xproj[:, n * _TK:(n + 1) * _TK] = (
            part + bg_ref[:, n * _TK:(n + 1) * _TK])

    whh_cp.wait()
    wout_cp.start()
    whh16[...] = whh_v[...].astype(jnp.bfloat16)

    h = h0_ref[...]
    c = c0_ref[...]
    for t in range(T):
        h16b = jnp.broadcast_to(h.astype(jnp.bfloat16), (16, H))
        base = 32 * (t % 2)
        for n in range(NX):
            mxu = n % 2
            addr = base + 8 * (n // 2)
            for k in range(KT):
                msr = msr_state[mxu]
                msr_state[mxu] = msr ^ 1
                pltpu.matmul_push_rhs(
                    whh16[k * _TK:(k + 1) * _TK,
                          n * _TK:(n + 1) * _TK], msr, mxu)
                pltpu.matmul_acc_lhs(addr, h16b[:, k * _TK:(k + 1) * _TK],
                                     mxu, load_staged_rhs=msr)
        gch = []
        for n in range(NX):
            part = pltpu.matmul_pop(base + 8 * (n // 2), (16, _TK),
                                    jnp.float32, n % 2)
            gch.append(part[0:1, :] + xproj[t:t + 1, n * _TK:(n + 1) * _TK])
        i_g = jax.nn.sigmoid(jnp.concatenate(gch[0:2], axis=1))
        f_g = jax.nn.sigmoid(jnp.concatenate(gch[2:4], axis=1))
        g_g = jnp.tanh(jnp.concatenate(gch[4:6], axis=1))
        o_g = jax.nn.sigmoid(jnp.concatenate(gch[6:8], axis=1))
        c = f_g * c + i_g * g_g
        h = o_g * jnp.tanh(c)
        hbuf[t:t + 1, :] = h

    h_out_ref[...] = h
    c_out_ref[...] = c

    wout_cp.wait()
    hb16 = hbuf[...].astype(jnp.bfloat16)
    NV = scores_hbm.shape[1] // _TK
    for n in range(NV):
        mm_tiles(hb16, wout_v, n, n % 2, 16 * (n // 2), T)
    out_cps = []
    for n in range(NV):
        part = pltpu.matmul_pop(16 * (n // 2), (T, _TK), jnp.float32, n % 2)
        slot = n % 2
        if n >= 2:
            out_cps[n - 2].wait()
        sbuf[slot, :, :] = jax.nn.sigmoid(
            part + bout_ref[:, n * _TK:(n + 1) * _TK])
        cp = pltpu.make_async_copy(
            sbuf.at[slot],
            scores_hbm.at[:, pl.ds(n * _TK, _TK)], sem_out.at[slot])
        cp.start()
        out_cps.append(cp)
    out_cps[NV - 2].wait()
    out_cps[NV - 1].wait()


@jax.jit
def kernel(words, emb, wih_t, whh_t, bg, wout_t, bout, h0, c0):
    V, E = emb.shape
    H = h0.shape[-1]
    T = words.shape[0]

    vmem = lambda i, w: (0, 0)
    any_spec = pl.BlockSpec(memory_space=pl.ANY)

    scores, h_out, c_out = pl.pallas_call(
        _lstm_kernel,
        out_shape=(
            jax.ShapeDtypeStruct((T, V), jnp.float32),
            jax.ShapeDtypeStruct((1, H), jnp.float32),
            jax.ShapeDtypeStruct((1, H), jnp.float32),
        ),
        grid_spec=pltpu.PrefetchScalarGridSpec(
            num_scalar_prefetch=1,
            grid=(1,),
            in_specs=[
                any_spec,
                any_spec,
                any_spec,
                any_spec,
                pl.BlockSpec((1, 4 * H), vmem),
                pl.BlockSpec((1, V), vmem),
                pl.BlockSpec((1, H), vmem),
                pl.BlockSpec((1, H), vmem),
            ],
            out_specs=[
                pl.BlockSpec(memory_space=pl.ANY),
                pl.BlockSpec((1, H), vmem),
                pl.BlockSpec((1, H), vmem),
            ],
            scratch_shapes=[
                pltpu.VMEM((T, E), jnp.float32),
                pltpu.VMEM((E, 4 * H), jnp.float32),
                pltpu.VMEM((H, 4 * H), jnp.float32),
                pltpu.VMEM((H, V), jnp.float32),
                pltpu.VMEM((H, 4 * H), jnp.bfloat16),
                pltpu.VMEM((T, 4 * H), jnp.float32),
                pltpu.VMEM((T, H), jnp.float32),
                pltpu.VMEM((2, T, _TK), jnp.float32),
                pltpu.SemaphoreType.DMA,
                pltpu.SemaphoreType.DMA,
                pltpu.SemaphoreType.DMA,
                pltpu.SemaphoreType.DMA,
                pltpu.SemaphoreType.DMA((2,)),
            ],
        ),
        compiler_params=pltpu.CompilerParams(
            dimension_semantics=("arbitrary",)),
    )(words, emb, wih_t, whh_t, wout_t, bg, bout,
      h0.reshape(1, H), c0.reshape(1, H))

    return scores, (h_out.reshape(1, 1, H), c_out.reshape(1, 1, H))


# speedup vs baseline: 1.1490x; 1.1490x over previous
import jax
import jax.numpy as jnp
from jax.experimental import pallas as pl
from jax.experimental.pallas import tpu as pltpu

_TK = 256


def _lstm_kernel(words_ref,
                 emb_hbm,
                 wih_hbm,
                 whh_hbm,
                 wout_hbm,
                 bg_ref,
                 bout_ref,
                 h0_ref,
                 c0_ref,
                 scores_ref,
                 h_out_ref,
                 c_out_ref,
                 xbuf,
                 wih_v,
                 whh_v,
                 wout_v,
                 whh16,
                 xproj,
                 hbuf,
                 sem_rows, sem_wih, sem_whh, sem_wout):
    T = scores_ref.shape[0]
    H = h0_ref.shape[-1]
    V = emb_hbm.shape[0]
    KT = H // _TK

    row_cps = [
        pltpu.make_async_copy(
            emb_hbm.at[pl.ds(jnp.minimum(jnp.maximum(words_ref[t], 0), V - 1),
                             1), :],
            xbuf.at[pl.ds(t, 1), :], sem_rows)
        for t in range(T)
    ]
    for cp in row_cps:
        cp.start()
    wih_cp = pltpu.make_async_copy(wih_hbm, wih_v, sem_wih)
    wih_cp.start()
    whh_cp = pltpu.make_async_copy(whh_hbm, whh_v, sem_whh)
    whh_cp.start()
    wout_cp = pltpu.make_async_copy(wout_hbm, wout_v, sem_wout)

    for cp in row_cps:
        cp.wait()

    xb16 = xbuf[...].astype(jnp.bfloat16)
    msr_state = [0, 0]

    def mm_tiles(lhs16_full, w_ref, n_tile, mxu, base_addr, m):
        for k in range(KT):
            tile = w_ref[k * _TK:(k + 1) * _TK,
                         n_tile * _TK:(n_tile + 1) * _TK].astype(jnp.bfloat16)
            msr = msr_state[mxu]
            msr_state[mxu] = msr ^ 1
            pltpu.matmul_push_rhs(tile, msr, mxu)
            pltpu.matmul_acc_lhs(base_addr,
                                 lhs16_full[:, k * _TK:(k + 1) * _TK],
                                 mxu, load_staged_rhs=msr)

    NX = 4 * H // _TK
    wih_cp.wait()
    for n in range(NX):
        mm_tiles(xb16, wih_v, n, n % 2, 16 * (n // 2), T)
    for n in range(NX):
        part = pltpu.matmul_pop(16 * (n // 2), (T, _TK), jnp.float32, n % 2)
        xproj[:, n * _TK:(n + 1) * _TK] = (
            part + bg_ref[:, n * _TK:(n + 1) * _TK])

    whh_cp.wait()
    wout_cp.start()
    whh16[...] = whh_v[...].astype(jnp.bfloat16)

    h = h0_ref[...]
    c = c0_ref[...]
    for t in range(T):
        h16b = jnp.broadcast_to(h.astype(jnp.bfloat16), (16, H))
        base = 32 * (t % 2)
        for n in range(NX):
            mxu = n % 2
            addr = base + 8 * (n // 2)
            for k in range(KT):
                msr = msr_state[mxu]
                msr_state[mxu] = msr ^ 1
                pltpu.matmul_push_rhs(
                    whh16[k * _TK:(k + 1) * _TK,
                          n * _TK:(n + 1) * _TK], msr, mxu)
                pltpu.matmul_acc_lhs(addr, h16b[:, k * _TK:(k + 1) * _TK],
                                     mxu, load_staged_rhs=msr)
        gch = []
        for n in range(NX):
            part = pltpu.matmul_pop(base + 8 * (n // 2), (16, _TK),
                                    jnp.float32, n % 2)
            gch.append(part[0:1, :] + xproj[t:t + 1, n * _TK:(n + 1) * _TK])
        i_g = jax.nn.sigmoid(jnp.concatenate(gch[0:2], axis=1))
        f_g = jax.nn.sigmoid(jnp.concatenate(gch[2:4], axis=1))
        g_g = jnp.tanh(jnp.concatenate(gch[4:6], axis=1))
        o_g = jax.nn.sigmoid(jnp.concatenate(gch[6:8], axis=1))
        c = f_g * c + i_g * g_g
        h = o_g * jnp.tanh(c)
        hbuf[t:t + 1, :] = h

    h_out_ref[...] = h
    c_out_ref[...] = c

    wout_cp.wait()
    hb16 = hbuf[...].astype(jnp.bfloat16)
    NV = scores_ref.shape[1] // _TK
    for n in range(NV):
        mm_tiles(hb16, wout_v, n, n % 2, 16 * (n // 2), T)
    for n in range(NV):
        part = pltpu.matmul_pop(16 * (n // 2), (T, _TK), jnp.float32, n % 2)
        scores_ref[:, n * _TK:(n + 1) * _TK] = jax.nn.sigmoid(
            part + bout_ref[:, n * _TK:(n + 1) * _TK])


@jax.jit
def kernel(words, emb, wih_t, whh_t, bg, wout_t, bout, h0, c0):
    V, E = emb.shape
    H = h0.shape[-1]
    T = words.shape[0]

    vmem = lambda i, w: (0, 0)
    any_spec = pl.BlockSpec(memory_space=pl.ANY)

    scores, h_out, c_out = pl.pallas_call(
        _lstm_kernel,
        out_shape=(
            jax.ShapeDtypeStruct((T, V), jnp.float32),
            jax.ShapeDtypeStruct((1, H), jnp.float32),
            jax.ShapeDtypeStruct((1, H), jnp.float32),
        ),
        grid_spec=pltpu.PrefetchScalarGridSpec(
            num_scalar_prefetch=1,
            grid=(1,),
            in_specs=[
                any_spec,
                any_spec,
                any_spec,
                any_spec,
                pl.BlockSpec((1, 4 * H), vmem),
                pl.BlockSpec((1, V), vmem),
                pl.BlockSpec((1, H), vmem),
                pl.BlockSpec((1, H), vmem),
            ],
            out_specs=[
                pl.BlockSpec((T, V), vmem),
                pl.BlockSpec((1, H), vmem),
                pl.BlockSpec((1, H), vmem),
            ],
            scratch_shapes=[
                pltpu.VMEM((T, E), jnp.float32),
                pltpu.VMEM((E, 4 * H), jnp.float32),
                pltpu.VMEM((H, 4 * H), jnp.float32),
                pltpu.VMEM((H, V), jnp.float32),
                pltpu.VMEM((H, 4 * H), jnp.bfloat16),
                pltpu.VMEM((T, 4 * H), jnp.float32),
                pltpu.VMEM((T, H), jnp.float32),
                pltpu.SemaphoreType.DMA,
                pltpu.SemaphoreType.DMA,
                pltpu.SemaphoreType.DMA,
                pltpu.SemaphoreType.DMA,
            ],
        ),
        compiler_params=pltpu.CompilerParams(
            dimension_semantics=("arbitrary",)),
    )(words, emb, wih_t, whh_t, wout_t, bg, bout,
      h0.reshape(1, H), c0.reshape(1, H))

    return scores, (h_out.reshape(1, 1, H), c_out.reshape(1, 1, H))
